# baseline (speedup 1.0000x reference)
import functools

import jax
import jax.numpy as jnp
import numpy as np
from jax import lax
from jax.experimental import pallas as pl
from jax.experimental.pallas import tpu as pltpu
from jax.experimental.pallas import tpu_sc as plsc

N_NODES = 10000
E_EDGES = 320000
N_VIEWS = 2
D_F = 128
N_HEADS = 4
D_HEAD = D_F // N_HEADS

_NC = 2
_NS = 16
_CHUNK = 128
_NCHUNK = -(-E_EDGES // _CHUNK)
_NCHUNK = -(-_NCHUNK // _NS) * _NS
_EPAD = _NCHUNK * _CHUNK
_CPS = _NCHUNK // _NS
_ACC_ROWS = N_NODES + 8


def _spmm_sc(feat2, srcs, dst2, zeros):
    rps = (N_NODES // _NS) // 8 * 8
    rem_base = _NS * rps
    rem = N_NODES - rem_base

    mesh = plsc.VectorSubcoreMesh(core_axis_name="c", subcore_axis_name="s")

    @functools.partial(
        pl.kernel,
        out_type=jax.ShapeDtypeStruct((_NC * N_NODES, D_F), jnp.float32),
        mesh=mesh,
        scratch_types=[
            pltpu.VMEM((_CHUNK,), jnp.int32),
            pltpu.VMEM((_CHUNK,), jnp.int32),
            pltpu.VMEM((_CHUNK,), jnp.int32),
            pltpu.VMEM((_CHUNK,), jnp.int32),
            pltpu.VMEM((_CHUNK, D_F), jnp.float32),
            pltpu.VMEM((_CHUNK, D_F), jnp.float32),
            pltpu.VMEM_SHARED((_ACC_ROWS, D_F), jnp.float32),
            pltpu.SemaphoreType.DMA,
            pltpu.SemaphoreType.DMA,
            pltpu.SemaphoreType.DMA,
            pltpu.SemaphoreType.DMA,
            pltpu.SemaphoreType.DMA,
            pltpu.SemaphoreType.DMA,
        ],
    )
    def k(feat_hbm, srcs_hbm, dst_hbm, zero_hbm, out_hbm,
          sidx0, sidx1, didx0, didx1, rows0, rows1, acc,
          semg0, semg1, semis0, semis1, semid0, semid1):
        c = lax.axis_index("c")
        s = lax.axis_index("s")
        cbase = (c * _EPAD + s * _CPS * _CHUNK)
        dbase = s * _CPS * _CHUNK

        def load_idx(i, sb, db, sems, semd):
            pltpu.async_copy(srcs_hbm.at[pl.ds(cbase + i * _CHUNK, _CHUNK)],
                             sb, sems)
            pltpu.async_copy(dst_hbm.at[pl.ds(dbase + i * _CHUNK, _CHUNK)],
                             db, semd)

        def wait_idx(i, sb, db, sems, semd):
            pltpu.make_async_copy(
                srcs_hbm.at[pl.ds(cbase + i * _CHUNK, _CHUNK)], sb,
                sems).wait()
            pltpu.make_async_copy(
                dst_hbm.at[pl.ds(dbase + i * _CHUNK, _CHUNK)], db,
                semd).wait()

        def gather(i_sb, rb, semg):
            pltpu.async_copy(feat_hbm.at[i_sb], rb, semg)

        def wait_gather(i_sb, rb, semg):
            pltpu.make_async_copy(feat_hbm.at[i_sb], rb, semg).wait()

        pltpu.sync_copy(zero_hbm.at[pl.ds(s * rps, rps)],
                        acc.at[pl.ds(s * rps, rps)])
        @pl.when(s == _NS - 1)
        def _zero_rem():
            pltpu.sync_copy(zero_hbm.at[pl.ds(rem_base, rem)],
                            acc.at[pl.ds(rem_base, rem)])
        plsc.subcore_barrier()

        def body(i, carry):
            pltpu.sync_copy(srcs_hbm.at[pl.ds(cbase + i * _CHUNK, _CHUNK)],
                            sidx0)
            pltpu.sync_copy(dst_hbm.at[pl.ds(dbase + i * _CHUNK, _CHUNK)],
                            didx0)
            pltpu.async_copy(feat_hbm.at[sidx0], rows0, semg0).wait()
            pltpu.sync_copy(rows0, acc.at[didx0], add=True)
            return carry

        lax.fori_loop(0, _CPS, body, 0)

        plsc.subcore_barrier()
        pltpu.sync_copy(acc.at[pl.ds(s * rps, rps)],
                        out_hbm.at[pl.ds(c * N_NODES + s * rps, rps)])
        @pl.when(s == _NS - 1)
        def _write_rem():
            pltpu.sync_copy(acc.at[pl.ds(rem_base, rem)],
                            out_hbm.at[pl.ds(c * N_NODES + rem_base, rem)])

    return k(feat2, srcs, dst2, zeros)


def _l2norm_body(x_ref, o_ref):
    x = x_ref[...]
    n = jnp.sqrt(jnp.sum(x * x, axis=1, keepdims=True))
    o_ref[...] = x / jnp.maximum(n, 1e-12)


def _l2norm_tc(x, blk=2000):
    n_rows = x.shape[0]
    return pl.pallas_call(
        _l2norm_body,
        grid=(n_rows // blk,),
        in_specs=[pl.BlockSpec((blk, D_F), lambda i: (i, 0))],
        out_specs=pl.BlockSpec((blk, D_F), lambda i: (i, 0)),
        out_shape=jax.ShapeDtypeStruct((n_rows, D_F), jnp.float32),
    )(x)


def _ln(x, g, b):
    m = jnp.mean(x, axis=1, keepdims=True)
    xc = x - m
    v = jnp.mean(xc * xc, axis=1, keepdims=True)
    return xc * lax.rsqrt(v + 1e-5) * g + b


def _l2n(x):
    n = jnp.sqrt(jnp.sum(x * x, axis=1, keepdims=True))
    return x / jnp.maximum(n, 1e-12)


def _fused_body(s1_ref, s2_ref, w1t_ref, b1_ref, g1_ref, bt1_ref,
                w2t_ref, b2_ref, g2_ref, bt2_ref, wint_ref, bin_ref,
                woutt_ref, bout_ref, lng_ref, lnb_ref, o_ref):
    f32 = jnp.float32
    h1 = jnp.maximum(_ln(jnp.dot(_l2n(s1_ref[...]), w1t_ref[...],
                                 preferred_element_type=f32) + b1_ref[...],
                         g1_ref[...], bt1_ref[...]), 0.0)
    h2 = jnp.maximum(_ln(jnp.dot(_l2n(s2_ref[...]), w2t_ref[...],
                                 preferred_element_type=f32) + b2_ref[...],
                         g2_ref[...], bt2_ref[...]), 0.0)
    qkv1 = jnp.dot(h1, wint_ref[...], preferred_element_type=f32) + bin_ref[...]
    qkv2 = jnp.dot(h2, wint_ref[...], preferred_element_type=f32) + bin_ref[...]
    q1, k1, v1 = qkv1[:, :D_F], qkv1[:, D_F:2 * D_F], qkv1[:, 2 * D_F:]
    q2, k2, v2 = qkv2[:, :D_F], qkv2[:, D_F:2 * D_F], qkv2[:, 2 * D_F:]
    rr = lax.broadcasted_iota(jnp.int32, (D_F, D_F), 0) // D_HEAD
    cc = lax.broadcasted_iota(jnp.int32, (D_F, D_F), 1) // D_HEAD
    hm = (rr == cc).astype(f32) * np.float32(1.0 / np.sqrt(D_HEAD))
    s11 = jnp.dot(q1 * k1, hm, preferred_element_type=f32)
    s12 = jnp.dot(q1 * k2, hm, preferred_element_type=f32)
    s21 = jnp.dot(q2 * k1, hm, preferred_element_type=f32)
    s22 = jnp.dot(q2 * k2, hm, preferred_element_type=f32)
    w11 = 1.0 / (1.0 + jnp.exp(s12 - s11))
    w21 = 1.0 / (1.0 + jnp.exp(s22 - s21))
    a1 = w11 * v1 + (1.0 - w11) * v2
    a2 = w21 * v1 + (1.0 - w21) * v2
    o1 = jnp.dot(a1, woutt_ref[...], preferred_element_type=f32) + bout_ref[...]
    o2 = jnp.dot(a2, woutt_ref[...], preferred_element_type=f32) + bout_ref[...]
    y1 = _ln(o1 + h1, lng_ref[...], lnb_ref[...])
    y2 = _ln(o2 + h2, lng_ref[...], lnb_ref[...])
    o_ref[...] = (y1 + y2) * 0.5


def _fused_tc(s1, s2, w1t, b1, g1, bt1, w2t, b2, g2, bt2,
              wint, bin_, woutt, bout, lng, lnb, blk=2000):
    n_rows = s1.shape[0]

    def row_spec():
        return pl.BlockSpec((blk, D_F), lambda i: (i, 0))

    def full_spec(shape):
        return pl.BlockSpec(shape, lambda i: tuple(0 for _ in shape))

    return pl.pallas_call(
        _fused_body,
        grid=(n_rows // blk,),
        in_specs=[
            row_spec(), row_spec(),
            full_spec((D_F, D_F)), full_spec((1, D_F)), full_spec((1, D_F)),
            full_spec((1, D_F)),
            full_spec((D_F, D_F)), full_spec((1, D_F)), full_spec((1, D_F)),
            full_spec((1, D_F)),
            full_spec((D_F, 3 * D_F)), full_spec((1, 3 * D_F)),
            full_spec((D_F, D_F)), full_spec((1, D_F)),
            full_spec((1, D_F)), full_spec((1, D_F)),
        ],
        out_specs=pl.BlockSpec((blk, D_F), lambda i: (i, 0)),
        out_shape=jax.ShapeDtypeStruct((n_rows, D_F), jnp.float32),
    )(s1, s2, w1t, b1, g1, bt1, w2t, b2, g2, bt2, wint, bin_, woutt, bout,
      lng, lnb)


def kernel(feat_A, feat_P, edge_AP, edge_PA, W1, b1, g1, beta1, W2, b2, g2,
           beta2, attn_in_w, attn_in_b, attn_out_w, attn_out_b, ln_g, ln_b):
    fA = feat_A.reshape(N_VIEWS * N_NODES, D_F)
    fP = feat_P.reshape(N_VIEWS * N_NODES, D_F)
    zeros = jnp.zeros((N_NODES, D_F), jnp.float32)

    n_pad = _EPAD - E_EDGES
    pad_src = jnp.zeros((n_pad,), jnp.int32)
    pad_dst = jnp.full((n_pad,), N_NODES, jnp.int32)

    def mk_srcs(v0, v1):
        return jnp.concatenate([v0, pad_src, v1, pad_src])

    src_AP, src_PA = edge_AP[0], edge_PA[0]
    dst_AP = jnp.concatenate([edge_AP[1], pad_dst])
    dst_PA = jnp.concatenate([edge_PA[1], pad_dst])
    srcs_AP_nat = mk_srcs(src_AP * 2, src_AP * 2 + 1)
    srcs_PA_nat = mk_srcs(src_PA * 2, src_PA * 2 + 1)
    srcs_AP_vm = mk_srcs(src_AP, src_AP + N_NODES)

    s1 = _spmm_sc(fA, srcs_AP_nat, dst_AP, zeros)
    t = _spmm_sc(fP, srcs_PA_nat, dst_PA, zeros)
    tn = _l2norm_tc(t)
    s2 = _spmm_sc(tn, srcs_AP_vm, dst_AP, zeros)

    r2 = lambda v: v.reshape(1, -1)
    hP = _fused_tc(s1, s2, W1.T, r2(b1), r2(g1), r2(beta1),
                   W2.T, r2(b2), r2(g2), r2(beta2),
                   attn_in_w.T, r2(attn_in_b), attn_out_w.T, r2(attn_out_b),
                   r2(ln_g), r2(ln_b))
    h_P = hP.reshape(N_VIEWS, N_NODES, D_F).transpose(1, 0, 2)
    return feat_A, h_P


# speedup vs baseline: 1.3606x; 1.3606x over previous
import functools

import jax
import jax.numpy as jnp
import numpy as np
from jax import lax
from jax.experimental import pallas as pl
from jax.experimental.pallas import tpu as pltpu
from jax.experimental.pallas import tpu_sc as plsc

N_NODES = 10000
E_EDGES = 320000
N_VIEWS = 2
D_F = 128
N_HEADS = 4
D_HEAD = D_F // N_HEADS

_NC = 2
_NS = 16
_CHUNK = 128
_NCHUNK = -(-E_EDGES // _CHUNK)
_NCHUNK = -(-_NCHUNK // _NS) * _NS
_EPAD = _NCHUNK * _CHUNK
_CPS = _NCHUNK // _NS
_BLK = 8
_ACC_ROWS = N_NODES + 8


def _spmm_sc(feat2, srcs, dst2, zeros):
    rps = (N_NODES // _NS) // 8 * 8
    rem_base = _NS * rps
    rem = N_NODES - rem_base

    mesh = plsc.VectorSubcoreMesh(core_axis_name="c", subcore_axis_name="s")

    @functools.partial(
        pl.kernel,
        out_type=jax.ShapeDtypeStruct((_NC * N_NODES, D_F), jnp.float32),
        mesh=mesh,
        scratch_types=[
            pltpu.VMEM((_BLK, 1, _CHUNK), jnp.int32),
            pltpu.VMEM((_BLK, 1, _CHUNK), jnp.int32),
            pltpu.VMEM((_CHUNK, D_F), jnp.float32),
            pltpu.VMEM_SHARED((_ACC_ROWS, D_F), jnp.float32),
            pltpu.SemaphoreType.DMA,
        ],
    )
    def k(feat_hbm, srcs_hbm, dst_hbm, zero_hbm, out_hbm,
          sidx, didx, rows0, acc, semg0):
        c = lax.axis_index("c")
        s = lax.axis_index("s")
        crow = (c * _NCHUNK + s * _CPS)
        drow = s * _CPS

        pltpu.sync_copy(zero_hbm.at[pl.ds(s * rps, rps)],
                        acc.at[pl.ds(s * rps, rps)])
        @pl.when(s == _NS - 1)
        def _zero_rem():
            pltpu.sync_copy(zero_hbm.at[pl.ds(rem_base, rem)],
                            acc.at[pl.ds(rem_base, rem)])
        plsc.subcore_barrier()

        def body(b, carry):
            pltpu.sync_copy(srcs_hbm.at[pl.ds(crow + b * _BLK, _BLK)], sidx)
            pltpu.sync_copy(dst_hbm.at[pl.ds(drow + b * _BLK, _BLK)], didx)
            for j in range(_BLK):
                pltpu.async_copy(feat_hbm.at[sidx.at[j, 0]], rows0,
                                 semg0).wait()
                pltpu.sync_copy(rows0, acc.at[didx.at[j, 0]], add=True)
            return carry

        lax.fori_loop(0, _CPS // _BLK, body, 0)

        plsc.subcore_barrier()
        pltpu.sync_copy(acc.at[pl.ds(s * rps, rps)],
                        out_hbm.at[pl.ds(c * N_NODES + s * rps, rps)])
        @pl.when(s == _NS - 1)
        def _write_rem():
            pltpu.sync_copy(acc.at[pl.ds(rem_base, rem)],
                            out_hbm.at[pl.ds(c * N_NODES + rem_base, rem)])

    return k(feat2, srcs, dst2, zeros)


def _l2norm_body(x_ref, o_ref):
    x = x_ref[...]
    n = jnp.sqrt(jnp.sum(x * x, axis=1, keepdims=True))
    o_ref[...] = x / jnp.maximum(n, 1e-12)


def _l2norm_tc(x, blk=2000):
    n_rows = x.shape[0]
    return pl.pallas_call(
        _l2norm_body,
        grid=(n_rows // blk,),
        in_specs=[pl.BlockSpec((blk, D_F), lambda i: (i, 0))],
        out_specs=pl.BlockSpec((blk, D_F), lambda i: (i, 0)),
        out_shape=jax.ShapeDtypeStruct((n_rows, D_F), jnp.float32),
    )(x)


def _ln(x, g, b):
    m = jnp.mean(x, axis=1, keepdims=True)
    xc = x - m
    v = jnp.mean(xc * xc, axis=1, keepdims=True)
    return xc * lax.rsqrt(v + 1e-5) * g + b


def _l2n(x):
    n = jnp.sqrt(jnp.sum(x * x, axis=1, keepdims=True))
    return x / jnp.maximum(n, 1e-12)


def _fused_body(s1_ref, s2_ref, w1t_ref, b1_ref, g1_ref, bt1_ref,
                w2t_ref, b2_ref, g2_ref, bt2_ref, wint_ref, bin_ref,
                woutt_ref, bout_ref, lng_ref, lnb_ref, o_ref):
    f32 = jnp.float32
    h1 = jnp.maximum(_ln(jnp.dot(_l2n(s1_ref[...]), w1t_ref[...],
                                 preferred_element_type=f32) + b1_ref[...],
                         g1_ref[...], bt1_ref[...]), 0.0)
    h2 = jnp.maximum(_ln(jnp.dot(_l2n(s2_ref[...]), w2t_ref[...],
                                 preferred_element_type=f32) + b2_ref[...],
                         g2_ref[...], bt2_ref[...]), 0.0)
    qkv1 = jnp.dot(h1, wint_ref[...], preferred_element_type=f32) + bin_ref[...]
    qkv2 = jnp.dot(h2, wint_ref[...], preferred_element_type=f32) + bin_ref[...]
    q1, k1, v1 = qkv1[:, :D_F], qkv1[:, D_F:2 * D_F], qkv1[:, 2 * D_F:]
    q2, k2, v2 = qkv2[:, :D_F], qkv2[:, D_F:2 * D_F], qkv2[:, 2 * D_F:]
    rr = lax.broadcasted_iota(jnp.int32, (D_F, D_F), 0) // D_HEAD
    cc = lax.broadcasted_iota(jnp.int32, (D_F, D_F), 1) // D_HEAD
    hm = (rr == cc).astype(f32) * np.float32(1.0 / np.sqrt(D_HEAD))
    s11 = jnp.dot(q1 * k1, hm, preferred_element_type=f32)
    s12 = jnp.dot(q1 * k2, hm, preferred_element_type=f32)
    s21 = jnp.dot(q2 * k1, hm, preferred_element_type=f32)
    s22 = jnp.dot(q2 * k2, hm, preferred_element_type=f32)
    w11 = 1.0 / (1.0 + jnp.exp(s12 - s11))
    w21 = 1.0 / (1.0 + jnp.exp(s22 - s21))
    a1 = w11 * v1 + (1.0 - w11) * v2
    a2 = w21 * v1 + (1.0 - w21) * v2
    o1 = jnp.dot(a1, woutt_ref[...], preferred_element_type=f32) + bout_ref[...]
    o2 = jnp.dot(a2, woutt_ref[...], preferred_element_type=f32) + bout_ref[...]
    y1 = _ln(o1 + h1, lng_ref[...], lnb_ref[...])
    y2 = _ln(o2 + h2, lng_ref[...], lnb_ref[...])
    o_ref[...] = (y1 + y2) * 0.5


def _fused_tc(s1, s2, w1t, b1, g1, bt1, w2t, b2, g2, bt2,
              wint, bin_, woutt, bout, lng, lnb, blk=2000):
    n_rows = s1.shape[0]

    def row_spec():
        return pl.BlockSpec((blk, D_F), lambda i: (i, 0))

    def full_spec(shape):
        return pl.BlockSpec(shape, lambda i: tuple(0 for _ in shape))

    return pl.pallas_call(
        _fused_body,
        grid=(n_rows // blk,),
        in_specs=[
            row_spec(), row_spec(),
            full_spec((D_F, D_F)), full_spec((1, D_F)), full_spec((1, D_F)),
            full_spec((1, D_F)),
            full_spec((D_F, D_F)), full_spec((1, D_F)), full_spec((1, D_F)),
            full_spec((1, D_F)),
            full_spec((D_F, 3 * D_F)), full_spec((1, 3 * D_F)),
            full_spec((D_F, D_F)), full_spec((1, D_F)),
            full_spec((1, D_F)), full_spec((1, D_F)),
        ],
        out_specs=pl.BlockSpec((blk, D_F), lambda i: (i, 0)),
        out_shape=jax.ShapeDtypeStruct((n_rows, D_F), jnp.float32),
    )(s1, s2, w1t, b1, g1, bt1, w2t, b2, g2, bt2, wint, bin_, woutt, bout,
      lng, lnb)


def kernel(feat_A, feat_P, edge_AP, edge_PA, W1, b1, g1, beta1, W2, b2, g2,
           beta2, attn_in_w, attn_in_b, attn_out_w, attn_out_b, ln_g, ln_b):
    fA = jnp.transpose(feat_A, (1, 0, 2)).reshape(N_VIEWS * N_NODES, D_F)
    fP = jnp.transpose(feat_P, (1, 0, 2)).reshape(N_VIEWS * N_NODES, D_F)
    zeros = jnp.zeros((N_NODES, D_F), jnp.float32)

    n_pad = _EPAD - E_EDGES
    pad_src = jnp.zeros((n_pad,), jnp.int32)
    pad_dst = jnp.full((n_pad,), N_NODES, jnp.int32)

    def mk_srcs(src):
        return jnp.concatenate(
            [src, pad_src, src + N_NODES, pad_src]).reshape(-1, 1, _CHUNK)

    src_AP, src_PA = edge_AP[0], edge_PA[0]
    dst_AP = jnp.concatenate([edge_AP[1], pad_dst]).reshape(-1, 1, _CHUNK)
    dst_PA = jnp.concatenate([edge_PA[1], pad_dst]).reshape(-1, 1, _CHUNK)
    srcs_AP = mk_srcs(src_AP)
    srcs_PA = mk_srcs(src_PA)

    s1 = _spmm_sc(fA, srcs_AP, dst_AP, zeros)
    t = _spmm_sc(fP, srcs_PA, dst_PA, zeros)
    tn = _l2norm_tc(t)
    s2 = _spmm_sc(tn, srcs_AP, dst_AP, zeros)

    r2 = lambda v: v.reshape(1, -1)
    hP = _fused_tc(s1, s2, W1.T, r2(b1), r2(g1), r2(beta1),
                   W2.T, r2(b2), r2(g2), r2(beta2),
                   attn_in_w.T, r2(attn_in_b), attn_out_w.T, r2(attn_out_b),
                   r2(ln_g), r2(ln_b))
    h_P = hP.reshape(N_VIEWS, N_NODES, D_F).transpose(1, 0, 2)
    return feat_A, h_P
